# baseline (speedup 1.0000x reference)
import math

import jax
import jax.numpy as jnp
from jax import lax
from jax.experimental import pallas as pl
from jax.experimental.pallas import tpu as pltpu

_M = 0.2
_S = 30.0
_COS_M = math.cos(_M)
_SIN_M = math.sin(_M)
_TH = math.cos(math.pi - _M)
_MM = math.sin(math.pi - _M) * _M

_BR = 1024
_BC = 2048


def _body(lab_ref, cos_ref, out_ref):
    j = pl.program_id(1)
    x = cos_ref[...]
    lab_loc = lab_ref[...] - j * _BC
    col = lax.broadcasted_iota(jnp.int32, x.shape, 1)
    m = col == lab_loc
    v = jnp.sum(jnp.where(m, x, 0.0), axis=1, keepdims=True)
    sine = jnp.sqrt(jnp.clip(1.0 - v * v, 0.0, 1.0))
    phi = v * _COS_M - sine * _SIN_M
    phi = jnp.where(v - _TH > 0, phi, v - _MM)
    out_ref[...] = jnp.where(m, _S * phi, _S * x)


def kernel(cosine, label):
    n, v = cosine.shape
    lab2d = label.astype(jnp.int32).reshape(n, 1)
    grid = (n // _BR, pl.cdiv(v, _BC))
    return pl.pallas_call(
        _body,
        grid=grid,
        in_specs=[
            pl.BlockSpec((_BR, 1), lambda i, j: (i, 0)),
            pl.BlockSpec((_BR, _BC), lambda i, j: (i, j)),
        ],
        out_specs=pl.BlockSpec((_BR, _BC), lambda i, j: (i, j)),
        out_shape=jax.ShapeDtypeStruct((n, v), jnp.float32),
        compiler_params=pltpu.CompilerParams(
            dimension_semantics=("parallel", "parallel"),
        ),
    )(lab2d, cosine)
